# baseline (speedup 1.0000x reference)
import jax
import jax.numpy as jnp
from jax import lax
from jax.experimental import pallas as pl
from jax.experimental.pallas import tpu as pltpu
from jax.experimental.pallas import tpu_sc as plsc

NUM_FRAMES = 16 * 4096
NUM_CLASSES_PS = 64
OUT_W = 7
LANES = 16
BF = 256


def _sc_powerset(x_flat, map_flat):
    mesh = plsc.VectorSubcoreMesh(core_axis_name="c", subcore_axis_name="s")
    grid = NUM_FRAMES // BF

    @pl.kernel(
        out_type=jax.ShapeDtypeStruct((NUM_FRAMES * OUT_W,), jnp.float32),
        mesh=mesh,
        scratch_types=[
            pltpu.VMEM((NUM_CLASSES_PS * OUT_W,), jnp.float32),
            pltpu.SemaphoreType.DMA,
        ],
        compiler_params=pltpu.CompilerParams(needs_layout_passes=False),
    )
    def k(x_hbm, map_hbm, out_hbm, map_v, sem):
        pltpu.async_copy(map_hbm, map_v, sem).wait()

        iota = jnp.arange(LANES, dtype=jnp.int32)
        iota64 = iota * NUM_CLASSES_PS
        iota7 = iota * OUT_W

        def body(in_v, out_v):
            @pl.loop(0, BF // LANES)
            def _(g):
                ibase = g * (LANES * NUM_CLASSES_PS)
                idx0 = iota64 + ibase
                best = plsc.load_gather(in_v, [idx0])
                bidx7 = jnp.zeros((LANES,), jnp.int32)
                for c in range(1, NUM_CLASSES_PS):
                    v = plsc.load_gather(in_v, [idx0 + c])
                    m = v > best
                    best = jnp.where(m, v, best)
                    bidx7 = jnp.where(m, c * OUT_W, bidx7)
                oidx0 = iota7 + g * (LANES * OUT_W)
                for j in range(OUT_W):
                    vals = plsc.load_gather(map_v, [bidx7 + j])
                    plsc.store_scatter(out_v, [oidx0 + j], vals)

        pltpu.emit_pipeline(
            body,
            grid=(grid,),
            in_specs=[
                pl.BlockSpec((BF * NUM_CLASSES_PS,), lambda i: (i,)),
            ],
            out_specs=[
                pl.BlockSpec((BF * OUT_W,), lambda i: (i,)),
            ],
            core_axis_name=("c", "s"),
            dimension_semantics=(pltpu.PARALLEL,),
        )(x_hbm, out_hbm)

    return k(x_flat, map_flat)


@jax.jit
def kernel(powerset, mapping):
    nb, nf, _ = powerset.shape
    x_flat = powerset.reshape(-1)
    map_flat = mapping.reshape(-1)
    out = _sc_powerset(x_flat, map_flat)
    return out.reshape(nb, nf, OUT_W)


# speedup vs baseline: 1.1172x; 1.1172x over previous
import jax
import jax.numpy as jnp
from jax import lax
from jax.experimental import pallas as pl
from jax.experimental.pallas import tpu as pltpu
from jax.experimental.pallas import tpu_sc as plsc

NUM_FRAMES = 16 * 4096
NUM_CLASSES_PS = 64
OUT_W = 7
LANES = 16
BF = 256


def _sc_powerset(x_flat, map_flat):
    mesh = plsc.VectorSubcoreMesh(core_axis_name="c", subcore_axis_name="s")
    grid = NUM_FRAMES // BF

    @pl.kernel(
        out_type=jax.ShapeDtypeStruct((NUM_FRAMES * OUT_W,), jnp.float32),
        mesh=mesh,
        scratch_types=[
            pltpu.VMEM((NUM_CLASSES_PS * OUT_W,), jnp.float32),
            pltpu.SemaphoreType.DMA,
        ],
        compiler_params=pltpu.CompilerParams(needs_layout_passes=False),
    )
    def k(x_hbm, map_hbm, out_hbm, map_v, sem):
        pltpu.async_copy(map_hbm, map_v, sem).wait()

        iota = jnp.arange(LANES, dtype=jnp.int32)
        iota64 = iota * NUM_CLASSES_PS
        iota7 = iota * OUT_W

        NCHAIN = 8
        CLEN = NUM_CLASSES_PS // NCHAIN

        def body(in_v, out_v):
            @pl.loop(0, BF // LANES)
            def _(g):
                ibase = g * (LANES * NUM_CLASSES_PS)
                idx0 = iota64 + ibase
                bests, bidxs = [], []
                for k in range(NCHAIN):
                    c0 = k * CLEN
                    b = plsc.load_gather(in_v, [idx0 + c0])
                    bi7 = jnp.full((LANES,), c0 * OUT_W, jnp.int32)
                    for c in range(c0 + 1, c0 + CLEN):
                        v = plsc.load_gather(in_v, [idx0 + c])
                        m = v > b
                        b = jnp.where(m, v, b)
                        bi7 = jnp.where(m, c * OUT_W, bi7)
                    bests.append(b)
                    bidxs.append(bi7)
                best, bidx7 = bests[0], bidxs[0]
                for k in range(1, NCHAIN):
                    m = bests[k] > best
                    best = jnp.where(m, bests[k], best)
                    bidx7 = jnp.where(m, bidxs[k], bidx7)
                oidx0 = iota7 + g * (LANES * OUT_W)
                for j in range(OUT_W):
                    vals = plsc.load_gather(map_v, [bidx7 + j])
                    plsc.store_scatter(out_v, [oidx0 + j], vals)

        pltpu.emit_pipeline(
            body,
            grid=(grid,),
            in_specs=[
                pl.BlockSpec((BF * NUM_CLASSES_PS,), lambda i: (i,)),
            ],
            out_specs=[
                pl.BlockSpec((BF * OUT_W,), lambda i: (i,)),
            ],
            core_axis_name=("c", "s"),
            dimension_semantics=(pltpu.PARALLEL,),
        )(x_hbm, out_hbm)

    return k(x_flat, map_flat)


@jax.jit
def kernel(powerset, mapping):
    nb, nf, _ = powerset.shape
    x_flat = powerset.reshape(-1)
    map_flat = mapping.reshape(-1)
    out = _sc_powerset(x_flat, map_flat)
    return out.reshape(nb, nf, OUT_W)


# speedup vs baseline: 5.8676x; 5.2521x over previous
import jax
import jax.numpy as jnp
from jax.experimental import pallas as pl
from jax.experimental.pallas import tpu as pltpu
from jax.experimental.pallas import tpu_sc as plsc

NB = 16
NF = 4096
NC = 64
OUT_W = 7
LANES = 16
CF = 256
NCHAIN = 8
CLEN = NC // NCHAIN


def _sc_powerset(x2d, map_flat):
    mesh = plsc.VectorSubcoreMesh(core_axis_name="c", subcore_axis_name="s")

    @pl.kernel(
        out_type=jax.ShapeDtypeStruct((OUT_W, NB, NF), jnp.float32),
        mesh=mesh,
        scratch_types=[
            pltpu.VMEM((NC * OUT_W,), jnp.float32),
            pltpu.SemaphoreType.DMA,
        ],
        compiler_params=pltpu.CompilerParams(
            needs_layout_passes=False, skip_device_barrier=True
        ),
    )
    def k(x_hbm, map_hbm, out_hbm, map_v, sem):
        pltpu.async_copy(map_hbm, map_v, sem).wait()

        def body(in_v, out_v):
            @pl.loop(0, CF // LANES)
            def _(g):
                sl = pl.ds(g * LANES, LANES)
                bests, bidxs = [], []
                for kk in range(NCHAIN):
                    c0 = kk * CLEN
                    bv = in_v[c0, sl]
                    bi = jnp.full((LANES,), c0, jnp.int32)
                    for c in range(c0 + 1, c0 + CLEN):
                        v = in_v[c, sl]
                        m = v > bv
                        bv = jnp.where(m, v, bv)
                        bi = jnp.where(m, c, bi)
                    bests.append(bv)
                    bidxs.append(bi)
                best, bidx = bests[0], bidxs[0]
                for kk in range(1, NCHAIN):
                    m = bests[kk] > best
                    best = jnp.where(m, bests[kk], best)
                    bidx = jnp.where(m, bidxs[kk], bidx)
                for j in range(OUT_W):
                    vals = plsc.load_gather(map_v, [bidx + j * NC])
                    out_v[j, 0, sl] = vals

        ncol = NF // CF
        pltpu.emit_pipeline(
            body,
            grid=(NB * ncol,),
            in_specs=[
                pl.BlockSpec((NC, CF), lambda i: (i // ncol, i % ncol)),
            ],
            out_specs=[
                pl.BlockSpec((OUT_W, 1, CF), lambda i: (0, i // ncol, i % ncol)),
            ],
            core_axis_name=("c", "s"),
            dimension_semantics=(pltpu.PARALLEL,),
        )(x_hbm, out_hbm)

    return k(x2d, map_flat)


@jax.jit
def kernel(powerset, mapping):
    x2d = powerset.transpose(0, 2, 1).reshape(NB * NC, NF)
    map_flat = mapping.T.reshape(-1)
    out = _sc_powerset(x2d, map_flat)
    return out.transpose(1, 2, 0)
